# baseline (speedup 1.0000x reference)
import functools

import jax
import jax.numpy as jnp
from jax import lax
from jax.experimental import pallas as pl
from jax.experimental.pallas import tpu as pltpu
from jax.experimental.pallas import tpu_sc as plsc

U = 10000
P = 10000
E = 320000
D = 128
H = D // 2
NS = 16
B = 128
BPT = 160
BPG = 32
EP = NS * BPT * B
NB = EP // B
PP = 10240
RPT = PP // NS
FIN = 128


def _sc_body(xp, ep, upi, pui, zer,
             fcat,
             idxv, rb0, rb1, rb2, rb3,
             tbl, acc, gs0, gs1, gs2, gs3, ss0, ss1, ss2, ss3):
  c = lax.axis_index("c")
  s = lax.axis_index("s")
  rbs = (rb0, rb1, rb2, rb3)
  gss = (gs0, gs1, gs2, gs3)
  sss = (ss0, ss1, ss2, ss3)
  sl_tile = pl.ds(s * RPT, RPT)
  csl = pl.ds(c * H, H)

  def stage(idx3, table, accb):
    def gather(blk, rb, sem):
      pltpu.async_copy(table.at[idxv.at[blk, 0]], rb, sem)

    def gwait(blk, rb, sem):
      pltpu.make_async_copy(table.at[idxv.at[blk, 0]], rb, sem).wait()

    def sstart(blk, rb, sem):
      pltpu.async_copy(rb, accb.at[idxv.at[blk, 1]], sem, add=True)

    def swait(blk, rb, sem):
      pltpu.make_async_copy(rb, accb.at[idxv.at[blk, 1]], sem).wait()

    def scale(blk, rb):
      @plsc.parallel_loop(0, B // 16, unroll=2)
      def grp(g):
        vvec = plsc.bitcast(idxv[blk, 2, pl.ds(g * 16, 16)], jnp.float32)
        for j in range(16):
          v = vvec[j]
          i = g * 16 + j
          for k in range(H // 16):
            sl = pl.ds(k * 16, 16)
            rb[i, sl] = rb[i, sl] * v

    def group(g, _):
      gb = s * BPT + g * BPG
      pltpu.sync_copy(idx3.at[pl.ds(gb, BPG)], idxv)

      gather(0, rb0, gs0)
      gather(1, rb1, gs1)
      gwait(0, rb0, gs0)
      scale(0, rb0)
      sstart(0, rb0, ss0)
      gather(2, rb2, gs2)
      gwait(1, rb1, gs1)
      scale(1, rb1)
      sstart(1, rb1, ss1)
      gather(3, rb3, gs3)

      def quad(j, _):
        for t in range(4):
          blk = 4 * j + 2 + t
          bi = (2 + t) % 4
          ni = t
          gwait(blk, rbs[bi], gss[bi])
          scale(blk, rbs[bi])
          sstart(blk, rbs[bi], sss[bi])
          swait(blk - 2, rbs[ni], sss[ni])
          gather(blk + 2, rbs[ni], gss[ni])
        return 0

      lax.fori_loop(0, (BPG - 4) // 4, quad, 0)

      gwait(BPG - 2, rb2, gs2)
      scale(BPG - 2, rb2)
      sstart(BPG - 2, rb2, ss2)
      swait(BPG - 4, rb0, ss0)
      gwait(BPG - 1, rb3, gs3)
      scale(BPG - 1, rb3)
      sstart(BPG - 1, rb3, ss3)
      swait(BPG - 3, rb1, ss1)
      swait(BPG - 2, rb2, ss2)
      swait(BPG - 1, rb3, ss3)
      return 0

    lax.fori_loop(0, BPT // BPG, group, 0)

  pltpu.sync_copy(xp.at[sl_tile, csl], tbl.at[sl_tile])
  pltpu.sync_copy(zer.at[sl_tile], acc.at[sl_tile])
  plsc.subcore_barrier()

  stage(upi, tbl, acc)
  plsc.subcore_barrier()

  pltpu.sync_copy(zer.at[sl_tile], tbl.at[sl_tile])
  plsc.subcore_barrier()

  stage(pui, acc, tbl)
  plsc.subcore_barrier()

  def fin(r, _):
    r0 = s * RPT + r * FIN
    rsl = pl.ds(r0, FIN)
    pltpu.async_copy(tbl.at[rsl], rb0, gs0)
    pltpu.async_copy(xp.at[rsl, csl], rb1, gs1)
    pltpu.async_copy(ep.at[rsl, csl], rb2, gs2)
    pltpu.make_async_copy(tbl.at[rsl], rb0, gs0).wait()
    pltpu.make_async_copy(xp.at[rsl, csl], rb1, gs1).wait()
    pltpu.make_async_copy(ep.at[rsl, csl], rb2, gs2).wait()

    @plsc.parallel_loop(0, FIN, unroll=2)
    def row(i):
      for k in range(H // 16):
        sl = pl.ds(k * 16, 16)
        rb3[i, sl] = rb0[i, sl] - rb1[i, sl] + rb2[i, sl]

    pltpu.sync_copy(rb3, fcat.at[pl.ds(c * PP + r0, FIN)])
    return 0

  lax.fori_loop(0, RPT // FIN, fin, 0)


@jax.jit
def _run(xp, ep, upi, pui, zer):
  mesh = plsc.VectorSubcoreMesh(core_axis_name="c", subcore_axis_name="s",
                                num_cores=2, num_subcores=NS)
  f32 = jnp.float32
  i32 = jnp.int32
  return pl.kernel(
      _sc_body,
      out_type=jax.ShapeDtypeStruct((2 * PP, H), f32),
      mesh=mesh,
      compiler_params=pltpu.CompilerParams(use_tc_tiling_on_sc=False,
                                           needs_layout_passes=False),
      scratch_types=[
          pltpu.VMEM((BPG, 3, B), i32),
          pltpu.VMEM((B, H), f32),
          pltpu.VMEM((B, H), f32),
          pltpu.VMEM((B, H), f32),
          pltpu.VMEM((B, H), f32),
          pltpu.VMEM_SHARED((PP, H), f32),
          pltpu.VMEM_SHARED((PP, H), f32),
          pltpu.SemaphoreType.DMA,
          pltpu.SemaphoreType.DMA,
          pltpu.SemaphoreType.DMA,
          pltpu.SemaphoreType.DMA,
          pltpu.SemaphoreType.DMA,
          pltpu.SemaphoreType.DMA,
          pltpu.SemaphoreType.DMA,
          pltpu.SemaphoreType.DMA,
      ],
  )(xp, ep, upi, pui, zer)


def kernel(t, x, up_rows, up_cols, up_vals, pu_rows, pu_cols, pu_vals, e):
  del t
  i32 = jnp.int32
  f32 = jnp.float32
  rpad = jnp.zeros((PP - P, D), f32)
  xp = jnp.concatenate([x, rpad], axis=0)
  ep = jnp.concatenate([e, rpad], axis=0)

  npad = EP - E
  zi = jnp.zeros((npad,), i32)

  def prep(cols, rows, vals):
    c2 = jnp.concatenate([cols.astype(i32), zi]).reshape(NB, 1, B)
    r2 = jnp.concatenate([rows.astype(i32), zi]).reshape(NB, 1, B)
    v2 = jnp.concatenate(
        [jax.lax.bitcast_convert_type(vals, i32), zi]).reshape(NB, 1, B)
    return jnp.concatenate([c2, r2, v2], axis=1)

  upi = prep(up_cols, up_rows, up_vals)
  pui = prep(pu_cols, pu_rows, pu_vals)
  zer = jnp.zeros((PP, H), f32)

  fcat = _run(xp, ep, upi, pui, zer)
  return jnp.concatenate([fcat[:P], fcat[PP:PP + P]], axis=1)


# speedup vs baseline: 1.0129x; 1.0129x over previous
import functools

import jax
import jax.numpy as jnp
from jax import lax
from jax.experimental import pallas as pl
from jax.experimental.pallas import tpu as pltpu
from jax.experimental.pallas import tpu_sc as plsc

U = 10000
P = 10000
E = 320000
D = 128
H = D // 2
NS = 16
B = 128
BPT = 160
BPG = 40
EP = NS * BPT * B
NB = EP // B
PP = 10240
RPT = PP // NS
FIN = 128


def _sc_body(xp, ep, upi, pui, zer,
             fcat,
             idxv, rb0, rb1, rb2, rb3,
             tbl, acc, gs0, gs1, gs2, gs3, ss0, ss1, ss2, ss3):
  c = lax.axis_index("c")
  s = lax.axis_index("s")
  rbs = (rb0, rb1, rb2, rb3)
  gss = (gs0, gs1, gs2, gs3)
  sss = (ss0, ss1, ss2, ss3)
  sl_tile = pl.ds(s * RPT, RPT)
  csl = pl.ds(c * H, H)

  def stage(idx3, table, accb):
    def gather(blk, rb, sem):
      pltpu.async_copy(table.at[idxv.at[blk, 0]], rb, sem)

    def gwait(blk, rb, sem):
      pltpu.make_async_copy(table.at[idxv.at[blk, 0]], rb, sem).wait()

    def sstart(blk, rb, sem):
      pltpu.async_copy(rb, accb.at[idxv.at[blk, 1]], sem, add=True)

    def swait(blk, rb, sem):
      pltpu.make_async_copy(rb, accb.at[idxv.at[blk, 1]], sem).wait()

    def scale(blk, rb):
      @plsc.parallel_loop(0, B // 16, unroll=2)
      def grp(g):
        vvec = plsc.bitcast(idxv[blk, 2, pl.ds(g * 16, 16)], jnp.float32)
        for j in range(16):
          v = vvec[j]
          i = g * 16 + j
          for k in range(H // 16):
            sl = pl.ds(k * 16, 16)
            rb[i, sl] = rb[i, sl] * v

    def group(g, _):
      gb = s * BPT + g * BPG
      pltpu.sync_copy(idx3.at[pl.ds(gb, BPG)], idxv)

      gather(0, rb0, gs0)
      gather(1, rb1, gs1)
      gwait(0, rb0, gs0)
      scale(0, rb0)
      sstart(0, rb0, ss0)
      gather(2, rb2, gs2)
      gwait(1, rb1, gs1)
      scale(1, rb1)
      sstart(1, rb1, ss1)
      gather(3, rb3, gs3)

      def quad(j, _):
        for t in range(4):
          blk = 4 * j + 2 + t
          bi = (2 + t) % 4
          ni = t
          gwait(blk, rbs[bi], gss[bi])
          scale(blk, rbs[bi])
          sstart(blk, rbs[bi], sss[bi])
          swait(blk - 2, rbs[ni], sss[ni])
          gather(blk + 2, rbs[ni], gss[ni])
        return 0

      lax.fori_loop(0, (BPG - 4) // 4, quad, 0)

      gwait(BPG - 2, rb2, gs2)
      scale(BPG - 2, rb2)
      sstart(BPG - 2, rb2, ss2)
      swait(BPG - 4, rb0, ss0)
      gwait(BPG - 1, rb3, gs3)
      scale(BPG - 1, rb3)
      sstart(BPG - 1, rb3, ss3)
      swait(BPG - 3, rb1, ss1)
      swait(BPG - 2, rb2, ss2)
      swait(BPG - 1, rb3, ss3)
      return 0

    lax.fori_loop(0, BPT // BPG, group, 0)

  pltpu.sync_copy(xp.at[sl_tile, csl], tbl.at[sl_tile])
  pltpu.sync_copy(zer.at[sl_tile], acc.at[sl_tile])
  plsc.subcore_barrier()

  stage(upi, tbl, acc)
  plsc.subcore_barrier()

  pltpu.sync_copy(zer.at[sl_tile], tbl.at[sl_tile])
  plsc.subcore_barrier()

  stage(pui, acc, tbl)
  plsc.subcore_barrier()

  def fin(r, _):
    r0 = s * RPT + r * FIN
    rsl = pl.ds(r0, FIN)
    pltpu.async_copy(tbl.at[rsl], rb0, gs0)
    pltpu.async_copy(xp.at[rsl, csl], rb1, gs1)
    pltpu.async_copy(ep.at[rsl, csl], rb2, gs2)
    pltpu.make_async_copy(tbl.at[rsl], rb0, gs0).wait()
    pltpu.make_async_copy(xp.at[rsl, csl], rb1, gs1).wait()
    pltpu.make_async_copy(ep.at[rsl, csl], rb2, gs2).wait()

    @plsc.parallel_loop(0, FIN, unroll=2)
    def row(i):
      for k in range(H // 16):
        sl = pl.ds(k * 16, 16)
        rb3[i, sl] = rb0[i, sl] - rb1[i, sl] + rb2[i, sl]

    pltpu.sync_copy(rb3, fcat.at[pl.ds(c * PP + r0, FIN)])
    return 0

  lax.fori_loop(0, RPT // FIN, fin, 0)


@jax.jit
def _run(xp, ep, upi, pui, zer):
  mesh = plsc.VectorSubcoreMesh(core_axis_name="c", subcore_axis_name="s",
                                num_cores=2, num_subcores=NS)
  f32 = jnp.float32
  i32 = jnp.int32
  return pl.kernel(
      _sc_body,
      out_type=jax.ShapeDtypeStruct((2 * PP, H), f32),
      mesh=mesh,
      compiler_params=pltpu.CompilerParams(use_tc_tiling_on_sc=False,
                                           needs_layout_passes=False),
      scratch_types=[
          pltpu.VMEM((BPG, 3, B), i32),
          pltpu.VMEM((B, H), f32),
          pltpu.VMEM((B, H), f32),
          pltpu.VMEM((B, H), f32),
          pltpu.VMEM((B, H), f32),
          pltpu.VMEM_SHARED((PP, H), f32),
          pltpu.VMEM_SHARED((PP, H), f32),
          pltpu.SemaphoreType.DMA,
          pltpu.SemaphoreType.DMA,
          pltpu.SemaphoreType.DMA,
          pltpu.SemaphoreType.DMA,
          pltpu.SemaphoreType.DMA,
          pltpu.SemaphoreType.DMA,
          pltpu.SemaphoreType.DMA,
          pltpu.SemaphoreType.DMA,
      ],
  )(xp, ep, upi, pui, zer)


def kernel(t, x, up_rows, up_cols, up_vals, pu_rows, pu_cols, pu_vals, e):
  del t
  i32 = jnp.int32
  f32 = jnp.float32
  rpad = jnp.zeros((PP - P, D), f32)
  xp = jnp.concatenate([x, rpad], axis=0)
  ep = jnp.concatenate([e, rpad], axis=0)

  npad = EP - E
  zi = jnp.zeros((npad,), i32)

  def prep(cols, rows, vals):
    c2 = jnp.concatenate([cols.astype(i32), zi]).reshape(NB, 1, B)
    r2 = jnp.concatenate([rows.astype(i32), zi]).reshape(NB, 1, B)
    v2 = jnp.concatenate(
        [jax.lax.bitcast_convert_type(vals, i32), zi]).reshape(NB, 1, B)
    return jnp.concatenate([c2, r2, v2], axis=1)

  upi = prep(up_cols, up_rows, up_vals)
  pui = prep(pu_cols, pu_rows, pu_vals)
  zer = jnp.zeros((PP, H), f32)

  fcat = _run(xp, ep, upi, pui, zer)
  return jnp.concatenate([fcat[:P], fcat[PP:PP + P]], axis=1)
